# baseline (speedup 1.0000x reference)
import functools

import jax
import jax.numpy as jnp
from jax import lax
from jax.experimental import pallas as pl
from jax.experimental.pallas import tpu as pltpu
from jax.experimental.pallas import tpu_sc as plsc

_LW_BITS = 17
_LW = 1 << _LW_BITS


def _detile_body(x_ref, o_ref):
    o_ref[...] = x_ref[...].reshape(8 * _LW)


def _detile(xt, n_strips, n_windows):
    return pl.pallas_call(
        _detile_body,
        grid=(n_strips, n_windows),
        in_specs=[pl.BlockSpec((8, _LW), lambda g, w: (g, w))],
        out_specs=pl.BlockSpec((8 * _LW,), lambda g, w: (g * n_windows + w,)),
        out_shape=jax.ShapeDtypeStruct(
            (n_strips * n_windows * 8 * _LW,), jnp.float32),
    )(xt)


def _sc_gather(scratch, idx1d, n_cols, b):
    info = plsc.get_sparse_core_info()
    num_workers = info.num_cores * info.num_subcores
    lanes = info.num_lanes
    cols_per_tile = n_cols // num_workers

    mesh = plsc.VectorSubcoreMesh(core_axis_name="c", subcore_axis_name="s")

    @functools.partial(
        pl.kernel,
        mesh=mesh,
        out_type=jax.ShapeDtypeStruct((n_cols * b,), jnp.float32),
        scratch_types=(
            [pltpu.VMEM((b,), jnp.int32) for _ in range(cols_per_tile)]
            + [pltpu.VMEM((b,), jnp.float32) for _ in range(cols_per_tile)]
            + [pltpu.SemaphoreType.DMA]
        ),
    )
    def gather_kernel(scratch_hbm, idx_hbm, out_hbm, *refs):
        idx_vs = refs[:cols_per_tile]
        val_vs = refs[cols_per_tile:2 * cols_per_tile]
        sem = refs[2 * cols_per_tile]
        wid = lax.axis_index("s") * info.num_cores + lax.axis_index("c")
        mask = jnp.int32(_LW - 1)

        def col_base(j):
            c = wid * jnp.int32(cols_per_tile) + jnp.int32(j)
            g = lax.shift_right_logical(c, jnp.int32(3))
            r = c & jnp.int32(7)
            return c * jnp.int32(b), (g * jnp.int32(1 << 23)
                                      + r * jnp.int32(1 << _LW_BITS))

        for j in range(cols_per_tile):
            base, _ = col_base(j)
            pltpu.sync_copy(idx_hbm.at[pl.ds(base, b)], idx_vs[j])

        for j in range(cols_per_tile):
            _, cbase = col_base(j)
            idx_v = idx_vs[j]

            def body(_, o, idx_v=idx_v, cbase=cbase):
                v = idx_v[pl.ds(o, lanes)]
                w = lax.shift_right_logical(v, jnp.int32(_LW_BITS))
                idx_v[pl.ds(o, lanes)] = (
                    lax.shift_left(w, jnp.int32(20)) + (v & mask) + cbase)
                return o + jnp.int32(lanes)

            lax.fori_loop(0, b // lanes, body, jnp.int32(0))

        for j in range(cols_per_tile):
            pltpu.async_copy(scratch_hbm.at[idx_vs[j]], val_vs[j], sem)
        for j in range(cols_per_tile):
            pltpu.make_async_copy(scratch_hbm.at[idx_vs[j]], val_vs[j],
                                  sem).wait()
        for j in range(cols_per_tile):
            base, _ = col_base(j)
            pltpu.sync_copy(val_vs[j], out_hbm.at[pl.ds(base, b)])

    return gather_kernel(scratch, idx1d)


def kernel(x, dim, index, sparse_grad):
    del dim, sparse_grad
    n_rows, n_cols = x.shape
    b, c = index.shape
    xt = x.T
    idx1d = index.T.astype(jnp.int32).reshape(-1)
    n_strips = n_cols // 8
    n_windows = -(-n_rows // _LW)

    scratch = _detile(xt, n_strips, n_windows)
    out1d = _sc_gather(scratch, idx1d, n_cols, b)
    return out1d.reshape(c, b).T
